# Initial kernel scaffold; baseline (speedup 1.0000x reference)
import jax
import jax.numpy as jnp
from jax.experimental import pallas as pl


def kernel(AdID, AdvertiserID, Depth, Position, DescriptionID, user_id, QueryID, KeywordID, TitleID, TitleToken, QueryToken, AdIDList, emb_table):
    raise NotImplementedError("write your pallas kernel here")


# speedup vs baseline: 1.2147x; 1.2147x over previous
import jax
import jax.numpy as jnp
from jax import lax
from jax.experimental import pallas as pl
from jax.experimental.pallas import tpu as pltpu
from jax.experimental.pallas import tpu_sc as plsc

B = 4096
L_HIST = 50
T_TOK = 20
V = 1_000_000
D = 64
M = 100_000

NC, NS, LANES = 2, 16, 16
NW = NC * NS
ROWS_PER_W = B // NW
CHUNK = 16
NCHUNK = ROWS_PER_W // CHUNK
NDIRECT = 49
NSLOT = 50
IDX_PITCH = 64
HHALF = 8

SCALAR_SLOTS = ((0, 0), (1, 1), (2, 0), (3, 0), (4, 2), (5, 3), (6, 4), (7, 5), (8, 6))
TTOK_OFF = 7
QTOK_OFF = 8


def _sc_body(adid, adv, dep, pos, desc, uid, qid, kid, tid, ttok, qtok, adl,
             table, out, featb, ttkb, qtkb, adlb, idxs, idxd, obuf, hbuf,
             sem_g, sem_h):
    wid = lax.axis_index("s") * NC + lax.axis_index("c")
    base = wid * ROWS_PER_W
    iota = lax.iota(jnp.int32, LANES)

    for f, ref in enumerate((adid, adv, dep, pos, desc, uid, qid, kid, tid)):
        pltpu.sync_copy(ref.at[pl.ds(base, ROWS_PER_W)],
                        featb.at[pl.ds(f * ROWS_PER_W, ROWS_PER_W)])
    pltpu.sync_copy(ttok.at[pl.ds(base, ROWS_PER_W)], ttkb)
    pltpu.sync_copy(qtok.at[pl.ds(base, ROWS_PER_W)], qtkb)
    pltpu.sync_copy(adl.at[pl.ds(base, ROWS_PER_W)], adlb)

    def build_group(g, carry):
        rb = g * LANES
        for s, (fi, off) in enumerate(SCALAR_SLOTS):
            v = featb[pl.ds(fi * ROWS_PER_W + rb, LANES)]
            idxs[s, pl.ds(rb, LANES)] = v % M + off * M
        for t in range(T_TOK):
            v = plsc.load_gather(ttkb, [rb + iota, jnp.full((LANES,), t, jnp.int32)])
            idxs[9 + t, pl.ds(rb, LANES)] = v % M + TTOK_OFF * M
        for t in range(T_TOK):
            v = plsc.load_gather(qtkb, [rb + iota, jnp.full((LANES,), t, jnp.int32)])
            idxs[29 + t, pl.ds(rb, LANES)] = v % M + QTOK_OFF * M
        return carry

    lax.fori_loop(0, ROWS_PER_W // LANES, build_group, 0)

    def transp_group(g, carry):
        for l in range(LANES):
            r = g * LANES + l
            for k in range(IDX_PITCH // LANES):
                v = plsc.load_gather(
                    idxs, [k * LANES + iota, jnp.full((LANES,), 1, jnp.int32) * r])
                idxd[pl.ds(r * IDX_PITCH + k * LANES, LANES)] = v
        return carry

    lax.fori_loop(0, ROWS_PER_W // LANES, transp_group, 0)

    def chunk_body(c, carry):
        row0 = c * CHUNK
        handles = []
        for r in range(CHUNK):
            idxslice = idxd.at[pl.ds((row0 + r) * IDX_PITCH, NDIRECT)]
            cp = pltpu.make_async_copy(table.at[idxslice],
                                       obuf.at[pl.ds(r * NSLOT, NDIRECT)], sem_g)
            cp.start()
            handles.append(cp)
        for half in range(CHUNK // HHALF):
            hh = []
            for rr in range(HHALF):
                r = half * HHALF + rr
                cp = pltpu.make_async_copy(table.at[adlb.at[row0 + r]],
                                           hbuf.at[pl.ds(rr * L_HIST, L_HIST)], sem_h)
                cp.start()
                hh.append(cp)
            for cp in hh:
                cp.wait()
            for rr in range(HHALF):
                r = half * HHALF + rr

                def pool_j(j, acc):
                    return tuple(acc[k] + hbuf[rr * L_HIST + j, pl.ds(k * LANES, LANES)]
                                 for k in range(D // LANES))

                acc = lax.fori_loop(0, L_HIST, pool_j,
                                    tuple(jnp.zeros((LANES,), jnp.float32)
                                          for _ in range(D // LANES)))
                for k in range(D // LANES):
                    obuf[r * NSLOT + NDIRECT, pl.ds(k * LANES, LANES)] = (
                        acc[k] * (1.0 / L_HIST))
        for cp in handles:
            cp.wait()
        pltpu.sync_copy(obuf, out.at[pl.ds((base + row0) * NSLOT, CHUNK * NSLOT)])
        return carry

    lax.fori_loop(0, NCHUNK, chunk_body, 0)


def kernel(AdID, AdvertiserID, Depth, Position, DescriptionID, user_id,
           QueryID, KeywordID, TitleID, TitleToken, QueryToken, AdIDList, emb_table):
    i32 = jnp.int32
    args = [a.astype(i32) for a in
            (AdID, AdvertiserID, Depth, Position, DescriptionID, user_id,
             QueryID, KeywordID, TitleID, TitleToken, QueryToken, AdIDList)]
    mesh = plsc.VectorSubcoreMesh(core_axis_name="c", subcore_axis_name="s",
                                  num_cores=NC, num_subcores=NS)
    out = pl.kernel(
        _sc_body,
        out_type=jax.ShapeDtypeStruct((B * NSLOT, D), jnp.float32),
        mesh=mesh,
        compiler_params=pltpu.CompilerParams(needs_layout_passes=False,
                                             use_tc_tiling_on_sc=False),
        scratch_types=[
            pltpu.VMEM((9 * ROWS_PER_W,), i32),
            pltpu.VMEM((ROWS_PER_W, T_TOK), i32),
            pltpu.VMEM((ROWS_PER_W, T_TOK), i32),
            pltpu.VMEM((ROWS_PER_W, L_HIST), i32),
            pltpu.VMEM((IDX_PITCH, ROWS_PER_W), i32),
            pltpu.VMEM((ROWS_PER_W * IDX_PITCH,), i32),
            pltpu.VMEM((CHUNK * NSLOT, D), jnp.float32),
            pltpu.VMEM((HHALF * L_HIST, D), jnp.float32),
            pltpu.SemaphoreType.DMA,
            pltpu.SemaphoreType.DMA,
        ],
    )(*args, emb_table)
    return out.reshape(B, NSLOT * D)
